# Initial kernel scaffold; baseline (speedup 1.0000x reference)
import jax
import jax.numpy as jnp
from jax.experimental import pallas as pl


def kernel(x, edge_index, Wl1, bl1, Wr1, Wl2, bl2, Wr2, Wl3, bl3, Wr3):
    raise NotImplementedError("write your pallas kernel here")


# speedup vs baseline: 4.0682x; 4.0682x over previous
import functools

import jax
import jax.numpy as jnp
from jax import lax
from jax.experimental import pallas as pl
from jax.experimental.pallas import tpu as pltpu
from jax.experimental.pallas import tpu_sc as plsc

N = 10000
E = 320000
D = 128

NC = 2
NS = 16
NW = NC * NS
EPW = E // NW
K = 80
NCH = EPW // K
NP = 10240
RPT = NP // NS
RCH = 128
NRC = RPT // RCH

_mesh = plsc.VectorSubcoreMesh(core_axis_name="c", subcore_axis_name="s",
                               num_cores=NC, num_subcores=NS)


def _zero_rows(ref, nrows, ncols16):
    zeros = jnp.zeros((16,), jnp.float32)

    def body(r, _):
        for j in range(ncols16):
            ref[r, pl.ds(16 * j, 16)] = zeros
        return 0

    lax.fori_loop(0, nrows, body, 0)


def _sc_agg_body(h_hbm, src_hbm, dst_hbm, acc_out,
                 src_v, dst_v, rows_v, stage_v, acc_sh, sem):
    c = lax.axis_index("c")
    s = lax.axis_index("s")
    wid = s * NC + c

    _zero_rows(stage_v, RCH, D // 16)
    row0 = s * RPT
    for z in range(NRC):
        pltpu.sync_copy(stage_v, acc_sh.at[pl.ds(row0 + z * RCH, RCH)])
    plsc.subcore_barrier()

    ebase = wid * EPW

    def chunk(j, _):
        base = ebase + j * K
        pltpu.sync_copy(src_hbm.at[pl.ds(base, K)], src_v)
        pltpu.sync_copy(dst_hbm.at[pl.ds(base, K)], dst_v)
        pltpu.async_copy(h_hbm.at[src_v], rows_v, sem).wait()
        pltpu.sync_copy(rows_v, acc_sh.at[dst_v], add=True)
        return 0

    lax.fori_loop(0, NCH, chunk, 0)
    plsc.subcore_barrier()

    for z in range(NRC):
        r0 = row0 + z * RCH
        pltpu.sync_copy(acc_sh.at[pl.ds(r0, RCH)], stage_v)
        pltpu.sync_copy(stage_v, acc_out.at[c, pl.ds(r0, RCH)])


_sc_agg = pl.kernel(
    _sc_agg_body,
    out_type=jax.ShapeDtypeStruct((NC, NP, D), jnp.float32),
    mesh=_mesh,
    scratch_types=[
        pltpu.VMEM((K,), jnp.int32),
        pltpu.VMEM((K,), jnp.int32),
        pltpu.VMEM((K, D), jnp.float32),
        pltpu.VMEM((RCH, D), jnp.float32),
        pltpu.VMEM_SHARED((NP, D), jnp.float32),
        pltpu.SemaphoreType.DMA,
    ],
)


BN = 1000


def _tc_layer_body(act, p_ref, c_ref, x_ref, wl_ref, bl_ref, wr_ref, o_ref):
    cnt = c_ref[0, :, 0:1] + c_ref[1, :, 0:1]
    mean = (p_ref[0] + p_ref[1]) / jnp.maximum(cnt, 1.0)
    h = lax.dot_general(mean, wl_ref[...], (((1,), (1,)), ((), ())),
                        precision=lax.Precision.HIGHEST,
                        preferred_element_type=jnp.float32)
    h = h + bl_ref[...]
    h = h + lax.dot_general(x_ref[...], wr_ref[...], (((1,), (1,)), ((), ())),
                            precision=lax.Precision.HIGHEST,
                            preferred_element_type=jnp.float32)
    if act == "relu":
        h = jnp.maximum(h, 0.0)
    elif act == "logsoftmax":
        m = jnp.max(h, axis=1, keepdims=True)
        h = h - m
        h = h - jnp.log(jnp.sum(jnp.exp(h), axis=1, keepdims=True))
    o_ref[...] = h


def _tc_layer(p, cnt, x, Wl, bl, Wr, act):
    return pl.pallas_call(
        functools.partial(_tc_layer_body, act),
        grid=(N // BN,),
        in_specs=[
            pl.BlockSpec((NC, BN, D), lambda i: (0, i, 0)),
            pl.BlockSpec((NC, BN, D), lambda i: (0, i, 0)),
            pl.BlockSpec((BN, D), lambda i: (i, 0)),
            pl.BlockSpec((D, D), lambda i: (0, 0)),
            pl.BlockSpec((1, D), lambda i: (0, 0)),
            pl.BlockSpec((D, D), lambda i: (0, 0)),
        ],
        out_specs=pl.BlockSpec((BN, D), lambda i: (i, 0)),
        out_shape=jax.ShapeDtypeStruct((N, D), jnp.float32),
    )(p, cnt, x, Wl, bl.reshape(1, D), Wr)


def kernel(x, edge_index, Wl1, bl1, Wr1, Wl2, bl2, Wr2, Wl3, bl3, Wr3):
    src = edge_index[0].astype(jnp.int32)
    dst = edge_index[1].astype(jnp.int32)

    cnt = _sc_agg(jnp.ones((N, D), jnp.float32), src, dst)
    p1 = _sc_agg(x, src, dst)
    h1 = _tc_layer(p1, cnt, x, Wl1, bl1, Wr1, "relu")
    p2 = _sc_agg(h1, src, dst)
    h2 = _tc_layer(p2, cnt, h1, Wl2, bl2, Wr2, "relu")
    p3 = _sc_agg(h2, src, dst)
    return _tc_layer(p3, cnt, h2, Wl3, bl3, Wr3, "logsoftmax")


# speedup vs baseline: 10.4372x; 2.5656x over previous
import functools

import jax
import jax.numpy as jnp
from jax import lax
from jax.experimental import pallas as pl
from jax.experimental.pallas import tpu as pltpu
from jax.experimental.pallas import tpu_sc as plsc

N = 10000
E = 320000
D = 128

NC = 2
NS = 16
NW = NC * NS
EPW = E // NW
K = 80
NCH = EPW // K
NP = 10240
RPT = NP // NS
RCH = K
NRC = RPT // RCH

_mesh = plsc.VectorSubcoreMesh(core_axis_name="c", subcore_axis_name="s",
                               num_cores=NC, num_subcores=NS)


def _zero_rows(ref, nrows, ncols16):
    zeros = jnp.zeros((16,), jnp.float32)

    def body(r, _):
        for j in range(ncols16):
            ref[r, pl.ds(16 * j, 16)] = zeros
        return 0

    lax.fori_loop(0, nrows, body, 0)


def _zero_acc(stage_v, acc_sh, s):
    _zero_rows(stage_v, RCH, D // 16)
    row0 = s * RPT
    for z in range(NRC):
        pltpu.sync_copy(stage_v, acc_sh.at[pl.ds(row0 + z * RCH, RCH)])
    return row0


def _copy_out(stage_v, acc_sh, acc_out, c, row0):
    for z in range(NRC):
        r0 = row0 + z * RCH
        pltpu.sync_copy(acc_sh.at[pl.ds(r0, RCH)], stage_v)
        pltpu.sync_copy(stage_v, acc_out.at[c, pl.ds(r0, RCH)])


def _sc_agg_body(h_hbm, src_hbm, dst_hbm, acc_out,
                 src_l, dst_ka, dst_kb, rows_a, rows_b, acc_sh,
                 sem_a, sem_b, sem_da, sem_db):
    c = lax.axis_index("c")
    s = lax.axis_index("s")
    wid = s * NC + c

    row0 = _zero_acc(rows_a, acc_sh, s)
    pltpu.sync_copy(src_hbm.at[pl.ds(wid * EPW, EPW)], src_l)
    plsc.subcore_barrier()
    ebase = wid * EPW

    def g(j, buf, sem):
        return pltpu.async_copy(h_hbm.at[src_l.at[pl.ds(j * K, K)]], buf, sem)

    def gwait(j, buf, sem):
        pltpu.make_async_copy(h_hbm.at[src_l.at[pl.ds(j * K, K)]], buf, sem).wait()

    def d(j, buf, sem):
        return pltpu.async_copy(dst_hbm.at[pl.ds(ebase + j * K, K)], buf, sem)

    def dwait(j, buf, sem):
        pltpu.make_async_copy(dst_hbm.at[pl.ds(ebase + j * K, K)], buf, sem).wait()

    d(0, dst_ka, sem_da)
    g(0, rows_a, sem_a)

    def pair(t, _):
        j = 2 * t
        d(j + 1, dst_kb, sem_db)
        g(j + 1, rows_b, sem_b)
        gwait(j, rows_a, sem_a)
        dwait(j, dst_ka, sem_da)
        pltpu.sync_copy(rows_a, acc_sh.at[dst_ka], add=True)
        d(j + 2, dst_ka, sem_da)
        g(j + 2, rows_a, sem_a)
        gwait(j + 1, rows_b, sem_b)
        dwait(j + 1, dst_kb, sem_db)
        pltpu.sync_copy(rows_b, acc_sh.at[dst_kb], add=True)
        return 0

    lax.fori_loop(0, (NCH - 1) // 2, pair, 0)
    gwait(NCH - 1, rows_a, sem_a)
    dwait(NCH - 1, dst_ka, sem_da)
    pltpu.sync_copy(rows_a, acc_sh.at[dst_ka], add=True)
    plsc.subcore_barrier()

    _copy_out(rows_a, acc_sh, acc_out, c, row0)


def _sc_count_body(dst_hbm, acc_out, dst_ka, dst_kb, ones_v, acc_sh,
                   sem_da, sem_db):
    c = lax.axis_index("c")
    s = lax.axis_index("s")
    wid = s * NC + c

    row0 = _zero_acc(ones_v, acc_sh, s)
    ones16 = jnp.ones((16,), jnp.float32)

    def fill(r, _):
        for j in range(D // 16):
            ones_v[r, pl.ds(16 * j, 16)] = ones16
        return 0

    lax.fori_loop(0, K, fill, 0)
    plsc.subcore_barrier()
    ebase = wid * EPW

    def d(j, buf, sem):
        return pltpu.async_copy(dst_hbm.at[pl.ds(ebase + j * K, K)], buf, sem)

    def dwait(j, buf, sem):
        pltpu.make_async_copy(dst_hbm.at[pl.ds(ebase + j * K, K)], buf, sem).wait()

    d(0, dst_ka, sem_da)

    def pair(t, _):
        j = 2 * t
        d(j + 1, dst_kb, sem_db)
        dwait(j, dst_ka, sem_da)
        pltpu.sync_copy(ones_v, acc_sh.at[dst_ka], add=True)
        d(j + 2, dst_ka, sem_da)
        dwait(j + 1, dst_kb, sem_db)
        pltpu.sync_copy(ones_v, acc_sh.at[dst_kb], add=True)
        return 0

    lax.fori_loop(0, (NCH - 1) // 2, pair, 0)
    dwait(NCH - 1, dst_ka, sem_da)
    pltpu.sync_copy(ones_v, acc_sh.at[dst_ka], add=True)
    plsc.subcore_barrier()

    _copy_out(ones_v, acc_sh, acc_out, c, row0)


_sc_agg = pl.kernel(
    _sc_agg_body,
    out_type=jax.ShapeDtypeStruct((NC, NP, D), jnp.float32),
    mesh=_mesh,
    scratch_types=[
        pltpu.VMEM((EPW,), jnp.int32),
        pltpu.VMEM((K,), jnp.int32),
        pltpu.VMEM((K,), jnp.int32),
        pltpu.VMEM((K, D), jnp.float32),
        pltpu.VMEM((K, D), jnp.float32),
        pltpu.VMEM_SHARED((NP, D), jnp.float32),
        pltpu.SemaphoreType.DMA,
        pltpu.SemaphoreType.DMA,
        pltpu.SemaphoreType.DMA,
        pltpu.SemaphoreType.DMA,
    ],
)

_sc_count = pl.kernel(
    _sc_count_body,
    out_type=jax.ShapeDtypeStruct((NC, NP, D), jnp.float32),
    mesh=_mesh,
    scratch_types=[
        pltpu.VMEM((K,), jnp.int32),
        pltpu.VMEM((K,), jnp.int32),
        pltpu.VMEM((K, D), jnp.float32),
        pltpu.VMEM_SHARED((NP, D), jnp.float32),
        pltpu.SemaphoreType.DMA,
        pltpu.SemaphoreType.DMA,
    ],
)


BN = 1000


def _tc_layer_body(act, p_ref, c_ref, x_ref, wl_ref, bl_ref, wr_ref, o_ref):
    cnt = c_ref[0, :, 0:1] + c_ref[1, :, 0:1]
    mean = (p_ref[0] + p_ref[1]) / jnp.maximum(cnt, 1.0)
    h = lax.dot_general(mean, wl_ref[...], (((1,), (1,)), ((), ())),
                        precision=lax.Precision.HIGHEST,
                        preferred_element_type=jnp.float32)
    h = h + bl_ref[...]
    h = h + lax.dot_general(x_ref[...], wr_ref[...], (((1,), (1,)), ((), ())),
                            precision=lax.Precision.HIGHEST,
                            preferred_element_type=jnp.float32)
    if act == "relu":
        h = jnp.maximum(h, 0.0)
    elif act == "logsoftmax":
        m = jnp.max(h, axis=1, keepdims=True)
        h = h - m
        h = h - jnp.log(jnp.sum(jnp.exp(h), axis=1, keepdims=True))
    o_ref[...] = h


def _tc_layer(p, cnt, x, Wl, bl, Wr, act):
    return pl.pallas_call(
        functools.partial(_tc_layer_body, act),
        grid=(N // BN,),
        in_specs=[
            pl.BlockSpec((NC, BN, D), lambda i: (0, i, 0)),
            pl.BlockSpec((NC, BN, D), lambda i: (0, i, 0)),
            pl.BlockSpec((BN, D), lambda i: (i, 0)),
            pl.BlockSpec((D, D), lambda i: (0, 0)),
            pl.BlockSpec((1, D), lambda i: (0, 0)),
            pl.BlockSpec((D, D), lambda i: (0, 0)),
        ],
        out_specs=pl.BlockSpec((BN, D), lambda i: (i, 0)),
        out_shape=jax.ShapeDtypeStruct((N, D), jnp.float32),
    )(p, cnt, x, Wl, bl.reshape(1, D), Wr)


def kernel(x, edge_index, Wl1, bl1, Wr1, Wl2, bl2, Wr2, Wl3, bl3, Wr3):
    src = edge_index[0].astype(jnp.int32)
    dst = edge_index[1].astype(jnp.int32)

    cnt = _sc_count(dst)
    p1 = _sc_agg(x, src, dst)
    h1 = _tc_layer(p1, cnt, x, Wl1, bl1, Wr1, "relu")
    p2 = _sc_agg(h1, src, dst)
    h2 = _tc_layer(p2, cnt, h1, Wl2, bl2, Wr2, "relu")
    p3 = _sc_agg(h2, src, dst)
    return _tc_layer(p3, cnt, h2, Wl3, bl3, Wr3, "logsoftmax")
